# Initial kernel scaffold; baseline (speedup 1.0000x reference)
import jax
import jax.numpy as jnp
from jax.experimental import pallas as pl


def kernel(x, samples0, W1, b1, W2, b2):
    raise NotImplementedError("write your pallas kernel here")


# speedup vs baseline: 547.6120x; 547.6120x over previous
import functools

import jax
import jax.numpy as jnp
from jax import lax
from jax.experimental import pallas as pl
from jax.experimental.pallas import tpu as pltpu
from jax.experimental.pallas import tpu_sc as plsc

_B, _OBS, _H = 64, 32, 64
_N = 16384
_C = 128
_K = 128
_LOW, _HIGH = -8.0, 8.0
_TWO_PI = 6.283185307179586


def _u01(bits):
    b = lax.bitcast_convert_type(bits, jnp.uint32)
    f = lax.bitcast_convert_type(
        jnp.bitwise_or(jnp.right_shift(b, jnp.uint32(9)), jnp.uint32(0x3F800000)),
        jnp.float32)
    return f - 1.0


def _tc_body(seed, sigma, is_final, x_ref, w1_ref, b1_ref, wv_ref, src_ref, *rest):
    if is_final:
        (out_ref,) = rest
    elif sigma is None:
        cdf_ref, u_ref = rest
    else:
        a_ref, cdf_ref, u_ref = rest

    pid = pl.program_id(0)
    pltpu.prng_seed(seed, pid)

    cb = jnp.dot(x_ref[0], w1_ref[...], preferred_element_type=jnp.float32) \
        + b1_ref[...]

    a = src_ref[0]
    if sigma is not None:
        u1 = 1.0 - _u01(pltpu.prng_random_bits((_C, _K)))
        u2 = _u01(pltpu.prng_random_bits((_C, _K)))
        z = jnp.sqrt(-2.0 * jnp.log(u1)) * jnp.cos(_TWO_PI * u2)
        a = jnp.clip(a + sigma * z, _LOW, _HIGH)
        if not is_final:
            a_ref[0] = a

    e = jnp.zeros((_C, _K), jnp.float32)
    for h in range(_H):
        e = e + jnp.maximum(cb[0:1, h:h + 1] + a * wv_ref[0:1, h:h + 1], 0.0) \
            * wv_ref[1:2, h:h + 1]
    m = jnp.min(jnp.min(e, axis=1, keepdims=True), axis=0, keepdims=True)

    if is_final:
        flat = (lax.broadcasted_iota(jnp.int32, (_C, _K), 0) * _K
                + lax.broadcasted_iota(jnp.int32, (_C, _K), 1))
        cand = jnp.where(e == m, flat, jnp.int32(1 << 30))
        nstar = jnp.min(jnp.min(cand, axis=1, keepdims=True), axis=0, keepdims=True)
        val = jnp.sum(jnp.sum(jnp.where(flat == nstar, a, 0.0),
                              axis=1, keepdims=True), axis=0, keepdims=True)
        out_ref[...] = val.reshape(1, 1, 1)
        return

    p = jnp.exp(m - e)
    r0 = lax.broadcasted_iota(jnp.int32, (_C, _K), 0)
    c1 = lax.broadcasted_iota(jnp.int32, (_C, _K), 1)
    tri_incl = (r0 <= c1).astype(jnp.float32)
    tri_low = (c1 < r0).astype(jnp.float32)
    incl = jnp.dot(p, tri_incl, preferred_element_type=jnp.float32)
    s_col = incl[:, _K - 1:]
    coarse = jnp.dot(tri_low, s_col, preferred_element_type=jnp.float32)
    cdf_ref[0] = incl + coarse
    tot = jnp.sum(jnp.sum(p, axis=1, keepdims=True), axis=0, keepdims=True)
    u_ref[0] = _u01(pltpu.prng_random_bits((_C, _K))) * tot


def _mk_tc(seed, sigma, is_final):
    body = functools.partial(_tc_body, seed, sigma, is_final)
    in_specs = [
        pl.BlockSpec((1, 1, _OBS), lambda i: (i, 0, 0)),
        pl.BlockSpec((_OBS, _H), lambda i: (0, 0)),
        pl.BlockSpec((1, _H), lambda i: (0, 0)),
        pl.BlockSpec((2, _H), lambda i: (0, 0)),
        pl.BlockSpec((1, _C, _K), lambda i: (i, 0, 0)),
    ]
    big = jax.ShapeDtypeStruct((_B, _C, _K), jnp.float32)
    big_spec = pl.BlockSpec((1, _C, _K), lambda i: (i, 0, 0))
    if is_final:
        out_shape = jax.ShapeDtypeStruct((_B, 1, 1), jnp.float32)
        out_specs = pl.BlockSpec((1, 1, 1), lambda i: (i, 0, 0))
    elif sigma is None:
        out_shape = (big, big)
        out_specs = (big_spec, big_spec)
    else:
        out_shape = (big, big, big)
        out_specs = (big_spec, big_spec, big_spec)
    return pl.pallas_call(
        body,
        grid=(_B,),
        in_specs=in_specs,
        out_shape=out_shape,
        out_specs=out_specs,
    )


_tc0 = _mk_tc(11, None, False)
_tc1 = _mk_tc(22, 0.33, False)
_tc2 = _mk_tc(33, 0.165, False)
_tcf = _mk_tc(44, 0.0825, True)

_SC_MESH = plsc.VectorSubcoreMesh(
    core_axis_name="c", subcore_axis_name="s", num_cores=2, num_subcores=16)


@functools.partial(
    pl.kernel,
    mesh=_SC_MESH,
    compiler_params=pltpu.CompilerParams(needs_layout_passes=False),
    out_type=jax.ShapeDtypeStruct((_B, _N), jnp.float32),
    scratch_types=[
        pltpu.VMEM((_N,), jnp.float32),
        pltpu.VMEM((_N,), jnp.float32),
        pltpu.VMEM((_N,), jnp.float32),
    ],
)
def _sc_resample(cdf_hbm, samp_hbm, u_hbm, out_hbm, cdf_v, samp_v, u_v):
    wid = lax.axis_index("s") * 2 + lax.axis_index("c")
    for rep in range(2):
        b = wid * 2 + rep
        pltpu.sync_copy(cdf_hbm.at[b], cdf_v)
        pltpu.sync_copy(samp_hbm.at[b], samp_v)
        pltpu.sync_copy(u_hbm.at[b], u_v)

        def grp(g, carry):
            u = u_v[pl.ds(g * 16, 16)]
            pos = jnp.zeros((16,), jnp.int32)
            w = _N // 2
            while w >= 1:
                probe = pos + (w - 1)
                vals = plsc.load_gather(cdf_v, [probe])
                pos = jnp.where(vals < u, pos + w, pos)
                w //= 2
            val = plsc.load_gather(samp_v, [pos])
            u_v[pl.ds(g * 16, 16)] = val
            return carry

        lax.fori_loop(0, _N // 16, grp, jnp.int32(0))
        pltpu.sync_copy(u_v, out_hbm.at[b])


def kernel(x, samples0, W1, b1, W2, b2):
    s0 = samples0.reshape(_B, _N)
    w1o = W1[:_OBS]
    wv = jnp.stack([W1[_OBS], W2[:, 0]])
    b1r = b1.reshape(1, _H)

    x3 = x.reshape(_B, 1, _OBS)
    cdf, u = _tc0(x3, w1o, b1r, wv, s0.reshape(_B, _C, _K))
    g = _sc_resample(cdf.reshape(_B, _N), s0, u.reshape(_B, _N))
    a1, cdf, u = _tc1(x3, w1o, b1r, wv, g.reshape(_B, _C, _K))
    g = _sc_resample(cdf.reshape(_B, _N), a1.reshape(_B, _N), u.reshape(_B, _N))
    a2, cdf, u = _tc2(x3, w1o, b1r, wv, g.reshape(_B, _C, _K))
    g = _sc_resample(cdf.reshape(_B, _N), a2.reshape(_B, _N), u.reshape(_B, _N))
    return _tcf(x3, w1o, b1r, wv, g.reshape(_B, _C, _K)).reshape(_B, 1)
